# baseline (device time: 12963 ns/iter reference)
import jax
import jax.numpy as jnp
from jax import lax
from jax.experimental import pallas as pl
from jax.experimental.pallas import tpu as pltpu

N_Z = 4
BM = 512


def kernel(x, dy, gamma):
    m, d = x.shape
    n_blocks = m // BM

    def body(
        x_ref,
        dy_ref,
        out_ref,
        acc_ref,
        sum2_ref,
        buf1,
        buf2,
        send_sems,
        recv_sems,
    ):
        i = pl.program_id(0)
        my_x = lax.axis_index("x")
        my_y = lax.axis_index("y")
        my_z = lax.axis_index("z")
        p1z = jnp.bitwise_xor(my_z, 1)
        p2z = jnp.bitwise_xor(my_z, 2)

        @pl.when(i == 0)
        def _():
            barrier_sem = pltpu.get_barrier_semaphore()
            for tz in (p1z, p2z):
                pl.semaphore_signal(
                    barrier_sem,
                    inc=1,
                    device_id=(my_x, my_y, tz),
                    device_id_type=pl.DeviceIdType.MESH,
                )

        xv = x_ref[:, :]
        dyv = dy_ref[:, :]
        mu = jnp.mean(xv, axis=1, keepdims=True)
        xc = xv - mu
        var = jnp.mean(xc * xc, axis=1, keepdims=True)
        rstd = lax.rsqrt(var + 1e-5)
        dgamma = jnp.sum(dyv * (xc * rstd), axis=0, keepdims=True)
        dbeta = jnp.sum(dyv, axis=0, keepdims=True)
        part = jnp.concatenate([dgamma, dbeta], axis=0)

        @pl.when(i == 0)
        def _():
            acc_ref[:, :] = part

        @pl.when(i > 0)
        def _():
            acc_ref[:, :] = acc_ref[:, :] + part

        @pl.when(i == n_blocks - 1)
        def _():
            barrier_sem = pltpu.get_barrier_semaphore()
            pl.semaphore_wait(barrier_sem, 2)

            r1 = pltpu.make_async_remote_copy(
                src_ref=acc_ref,
                dst_ref=buf1,
                send_sem=send_sems.at[0],
                recv_sem=recv_sems.at[0],
                device_id=(my_x, my_y, p1z),
                device_id_type=pl.DeviceIdType.MESH,
            )
            r1.start()
            r1.wait_send()
            r1.wait_recv()
            sum2_ref[:, :] = acc_ref[:, :] + buf1[:, :]

            r2 = pltpu.make_async_remote_copy(
                src_ref=sum2_ref,
                dst_ref=buf2,
                send_sem=send_sems.at[1],
                recv_sem=recv_sems.at[1],
                device_id=(my_x, my_y, p2z),
                device_id_type=pl.DeviceIdType.MESH,
            )
            r2.start()
            r2.wait_send()
            r2.wait_recv()
            out_ref[:, :] = sum2_ref[:, :] + buf2[:, :]

    return pl.pallas_call(
        body,
        grid=(n_blocks,),
        out_shape=jax.ShapeDtypeStruct((2, d), jnp.float32),
        in_specs=[
            pl.BlockSpec((BM, d), lambda i: (i, 0)),
            pl.BlockSpec((BM, d), lambda i: (i, 0)),
        ],
        out_specs=pl.BlockSpec((2, d), lambda i: (0, 0)),
        scratch_shapes=[
            pltpu.VMEM((2, d), jnp.float32),
            pltpu.VMEM((2, d), jnp.float32),
            pltpu.VMEM((2, d), jnp.float32),
            pltpu.VMEM((2, d), jnp.float32),
            pltpu.SemaphoreType.DMA((2,)),
            pltpu.SemaphoreType.DMA((2,)),
        ],
        compiler_params=pltpu.CompilerParams(
            collective_id=0,
            dimension_semantics=("arbitrary",),
        ),
    )(x, dy)


# device time: 11710 ns/iter; 1.1070x vs baseline; 1.1070x over previous
import jax
import jax.numpy as jnp
from jax import lax
from jax.experimental import pallas as pl
from jax.experimental.pallas import tpu as pltpu

N_Z = 4
BM = 512


def kernel(x, dy, gamma):
    m, d = x.shape
    n_blocks = m // BM

    def body(x_ref, dy_ref, out_ref, acc_ref, comm_ref, send_sems, recv_sems):
        i = pl.program_id(0)
        my_x = lax.axis_index("x")
        my_y = lax.axis_index("y")
        my_z = lax.axis_index("z")

        @pl.when(i == 0)
        def _():
            barrier_sem = pltpu.get_barrier_semaphore()
            for dz in range(1, N_Z):
                tz = (my_z + dz) % N_Z
                pl.semaphore_signal(
                    barrier_sem,
                    inc=1,
                    device_id=(my_x, my_y, tz),
                    device_id_type=pl.DeviceIdType.MESH,
                )

        xv = x_ref[:, :]
        dyv = dy_ref[:, :]
        mu = jnp.mean(xv, axis=1, keepdims=True)
        xc = xv - mu
        var = jnp.mean(xc * xc, axis=1, keepdims=True)
        rstd = lax.rsqrt(var + 1e-5)
        dgamma = jnp.sum(dyv * (xc * rstd), axis=0, keepdims=True)
        dbeta = jnp.sum(dyv, axis=0, keepdims=True)
        part = jnp.concatenate([dgamma, dbeta], axis=0)

        @pl.when(i == 0)
        def _():
            acc_ref[:, :] = part

        @pl.when(i > 0)
        def _():
            acc_ref[:, :] = acc_ref[:, :] + part

        @pl.when(i == n_blocks - 1)
        def _():
            barrier_sem = pltpu.get_barrier_semaphore()
            pl.semaphore_wait(barrier_sem, N_Z - 1)
            rdmas = []
            for dz in range(1, N_Z):
                tz = (my_z + dz) % N_Z
                rdma = pltpu.make_async_remote_copy(
                    src_ref=acc_ref,
                    dst_ref=comm_ref.at[dz - 1],
                    send_sem=send_sems.at[dz - 1],
                    recv_sem=recv_sems.at[dz - 1],
                    device_id=(my_x, my_y, tz),
                    device_id_type=pl.DeviceIdType.MESH,
                )
                rdma.start()
                rdmas.append(rdma)
            total = acc_ref[:, :]
            for k, rdma in enumerate(rdmas):
                rdma.wait_recv()
                total = total + comm_ref[k]
            out_ref[:, :] = total
            for rdma in rdmas:
                rdma.wait_send()

    return pl.pallas_call(
        body,
        grid=(n_blocks,),
        out_shape=jax.ShapeDtypeStruct((2, d), jnp.float32),
        in_specs=[
            pl.BlockSpec((BM, d), lambda i: (i, 0)),
            pl.BlockSpec((BM, d), lambda i: (i, 0)),
        ],
        out_specs=pl.BlockSpec((2, d), lambda i: (0, 0)),
        scratch_shapes=[
            pltpu.VMEM((2, d), jnp.float32),
            pltpu.VMEM((N_Z - 1, 2, d), jnp.float32),
            pltpu.SemaphoreType.DMA((N_Z - 1,)),
            pltpu.SemaphoreType.DMA((N_Z - 1,)),
        ],
        compiler_params=pltpu.CompilerParams(
            collective_id=0,
            dimension_semantics=("arbitrary",),
        ),
    )(x, dy)
